# baseline (speedup 1.0000x reference)
import functools

import jax
import jax.numpy as jnp
from jax import lax
from jax.experimental import pallas as pl
from jax.experimental.pallas import tpu as pltpu
from jax.experimental.pallas import tpu_sc as plsc

NUM_SEGMENTS = 16
TOTAL_TOK = 32768
D = 128
LANES = 16
KREG = D // LANES

T_SC = 20480
NWORKERS = 32
ROWS_PER_W = T_SC // NWORKERS
CHUNK = 320
NCHUNK = ROWS_PER_W // CHUNK
BSTEPS = (ROWS_PER_W - 1).bit_length()

T_TC = TOTAL_TOK - T_SC
BLOCK_TOK = 2048
TC_GRID = T_TC // BLOCK_TOK
TC_OFF = T_SC // BLOCK_TOK


def _sc_body(flat_hbm, ids_hbm, psum_hbm,
             ids_v, b_ref, bx_ref, acc_ref, buf, sem0, sem1):
    cid = lax.axis_index("c")
    sid = lax.axis_index("s")
    wid = sid * 2 + cid
    base = wid * ROWS_PER_W

    sems = (sem0, sem1)
    copies = [None, None]
    for c in range(min(2, NCHUNK)):
        copies[c] = pltpu.make_async_copy(
            flat_hbm.at[pl.ds(base + c * CHUNK, CHUNK)], buf.at[c], sems[c])
        copies[c].start()

    pltpu.sync_copy(ids_hbm.at[pl.ds(base, ROWS_PER_W)], ids_v)

    def z_body(s, _):
        for k in range(KREG):
            acc_ref[s, pl.ds(LANES * k, LANES)] = jnp.zeros((LANES,), jnp.float32)
        return 0

    lax.fori_loop(0, NUM_SEGMENTS + 1, z_body, 0)

    lane = lax.iota(jnp.int32, LANES)

    def step(t, lohi):
        lo, hi = lohi
        mid = (lo + hi) >> 1
        vals = plsc.load_gather(ids_v, [mid])
        pred = vals < lane
        return (jnp.where(pred, mid + 1, lo), jnp.where(pred, hi, mid))

    b_v, _ = lax.fori_loop(
        0, BSTEPS, step,
        (jnp.zeros((LANES,), jnp.int32),
         jnp.full((LANES,), ROWS_PER_W, jnp.int32)))

    bx_ref[pl.ds(0, LANES)] = b_v
    bx_ref[pl.ds(LANES, LANES)] = jnp.full((LANES,), ROWS_PER_W, jnp.int32)
    b_next = plsc.load_gather(bx_ref, [lane + 1])
    acc_ref[NUM_SEGMENTS, pl.ds(0, LANES)] = (b_next - b_v).astype(jnp.float32)

    for s in range(NUM_SEGMENTS):
        b_ref[s] = b_v[s]
    b_ref[NUM_SEGMENTS] = jnp.int32(ROWS_PER_W)

    for c in range(NCHUNK):
        cb = c & 1
        copies[cb].wait()

        def seg_body(s, _):
            lo = jnp.maximum(b_ref[s], c * CHUNK)
            hi = jnp.minimum(b_ref[s + 1], (c + 1) * CHUNK)

            def row_body(i, vs):
                r = i - c * CHUNK
                return tuple(
                    vs[k] + buf[cb, r, pl.ds(LANES * k, LANES)]
                    for k in range(KREG))

            vs = lax.fori_loop(
                lo, hi, row_body,
                tuple(jnp.zeros((LANES,), jnp.float32) for _ in range(KREG)))

            @pl.when(hi > lo)
            def _flush():
                for k in range(KREG):
                    acc_ref[s, pl.ds(LANES * k, LANES)] += vs[k]

            return 0

        lax.fori_loop(0, NUM_SEGMENTS, seg_body, 0)

        if c + 2 < NCHUNK:
            copies[cb] = pltpu.make_async_copy(
                flat_hbm.at[pl.ds(base + (c + 2) * CHUNK, CHUNK)], buf.at[cb],
                sems[cb])
            copies[cb].start()

    pltpu.sync_copy(acc_ref, psum_hbm.at[wid])


_sc_call = functools.partial(
    pl.kernel,
    out_type=jax.ShapeDtypeStruct((NWORKERS, NUM_SEGMENTS + 1, D), jnp.float32),
    mesh=plsc.VectorSubcoreMesh(core_axis_name="c", subcore_axis_name="s"),
    compiler_params=pltpu.CompilerParams(needs_layout_passes=False),
    scratch_types=[
        pltpu.VMEM((ROWS_PER_W,), jnp.int32),
        pltpu.SMEM((NUM_SEGMENTS + 1,), jnp.int32),
        pltpu.VMEM((2 * LANES,), jnp.int32),
        pltpu.VMEM((NUM_SEGMENTS + 1, D), jnp.float32),
        pltpu.VMEM((2, CHUNK, D), jnp.float32),
        pltpu.SemaphoreType.DMA,
        pltpu.SemaphoreType.DMA,
    ],
)


def _tc_body(ids_ref, flat_ref, psum_ref, pcnt_ref, acc_sum, acc_cnt):
    i = pl.program_id(0)

    @pl.when(i == 0)
    def _zero():
        acc_sum[...] = jnp.zeros_like(acc_sum)
        acc_cnt[...] = jnp.zeros_like(acc_cnt)

    ids = ids_ref[0, 0, :]
    seg_iota = jax.lax.broadcasted_iota(jnp.int32, (BLOCK_TOK, NUM_SEGMENTS), 1)
    onehot = (ids[:, None] == seg_iota).astype(jnp.float32)
    acc_sum[...] += jax.lax.dot_general(
        onehot, flat_ref[...],
        dimension_numbers=(((0,), (0,)), ((), ())),
        preferred_element_type=jnp.float32,
    )
    pcnt = jnp.sum(onehot, axis=0)
    acc_cnt[...] += jnp.broadcast_to(pcnt[:, None], (NUM_SEGMENTS, D))

    @pl.when(i == TC_GRID - 1)
    def _finish():
        psum_ref[...] = acc_sum[...]
        pcnt_ref[...] = acc_cnt[...]


def _combine_body(ps_ref, tps_ref, tpc_ref, out_ref):
    acc = jnp.sum(ps_ref[...], axis=0)
    sums = acc[:NUM_SEGMENTS, :] + tps_ref[...]
    cnts = acc[NUM_SEGMENTS, :NUM_SEGMENTS][:, None] + tpc_ref[...]
    out_ref[...] = sums / jnp.maximum(cnts, 1.0)


def kernel(flat, segment_ids):
    ids32 = segment_ids.astype(jnp.int32)
    psum = _sc_call(_sc_body)(flat, ids32)
    ids3 = ids32.reshape(TOTAL_TOK // BLOCK_TOK, 1, BLOCK_TOK)
    flat = pltpu.with_memory_space_constraint(flat, pltpu.MemorySpace.HBM)
    ids3 = pltpu.with_memory_space_constraint(ids3, pltpu.MemorySpace.HBM)
    tpsum, tpcnt = pl.pallas_call(
        _tc_body,
        grid=(TC_GRID,),
        in_specs=[
            pl.BlockSpec((1, 1, BLOCK_TOK), lambda i: (TC_OFF + i, 0, 0)),
            pl.BlockSpec((BLOCK_TOK, D), lambda i: (TC_OFF + i, 0)),
        ],
        out_specs=[
            pl.BlockSpec((NUM_SEGMENTS, D), lambda i: (0, 0)),
            pl.BlockSpec((NUM_SEGMENTS, D), lambda i: (0, 0)),
        ],
        out_shape=[
            jax.ShapeDtypeStruct((NUM_SEGMENTS, D), jnp.float32),
            jax.ShapeDtypeStruct((NUM_SEGMENTS, D), jnp.float32),
        ],
        scratch_shapes=[
            pltpu.VMEM((NUM_SEGMENTS, D), jnp.float32),
            pltpu.VMEM((NUM_SEGMENTS, D), jnp.float32),
        ],
    )(ids3, flat)
    return pl.pallas_call(
        _combine_body,
        out_shape=jax.ShapeDtypeStruct((NUM_SEGMENTS, D), jnp.float32),
    )(psum, tpsum, tpcnt)


# speedup vs baseline: 1.0208x; 1.0208x over previous
import functools

import jax
import jax.numpy as jnp
from jax import lax
from jax.experimental import pallas as pl
from jax.experimental.pallas import tpu as pltpu
from jax.experimental.pallas import tpu_sc as plsc

NUM_SEGMENTS = 16
TOTAL_TOK = 32768
D = 128
LANES = 16
KREG = D // LANES

T_SC = 18432
NWORKERS = 32
ROWS_PER_W = T_SC // NWORKERS
CHUNK = 288
NCHUNK = ROWS_PER_W // CHUNK
BSTEPS = (ROWS_PER_W - 1).bit_length()

T_TC = TOTAL_TOK - T_SC
BLOCK_TOK = 2048
TC_GRID = T_TC // BLOCK_TOK
TC_OFF = T_SC // BLOCK_TOK


def _sc_body(flat_hbm, ids_hbm, psum_hbm,
             ids_v, b_ref, bx_ref, acc_ref, buf, sem0, sem1):
    cid = lax.axis_index("c")
    sid = lax.axis_index("s")
    wid = sid * 2 + cid
    base = wid * ROWS_PER_W

    sems = (sem0, sem1)
    copies = [None, None]
    for c in range(min(2, NCHUNK)):
        copies[c] = pltpu.make_async_copy(
            flat_hbm.at[pl.ds(base + c * CHUNK, CHUNK)], buf.at[c], sems[c])
        copies[c].start()

    pltpu.sync_copy(ids_hbm.at[pl.ds(base, ROWS_PER_W)], ids_v)

    def z_body(s, _):
        for k in range(KREG):
            acc_ref[s, pl.ds(LANES * k, LANES)] = jnp.zeros((LANES,), jnp.float32)
        return 0

    lax.fori_loop(0, NUM_SEGMENTS + 1, z_body, 0)

    lane = lax.iota(jnp.int32, LANES)

    def step(t, lohi):
        lo, hi = lohi
        mid = (lo + hi) >> 1
        vals = plsc.load_gather(ids_v, [mid])
        pred = vals < lane
        return (jnp.where(pred, mid + 1, lo), jnp.where(pred, hi, mid))

    b_v, _ = lax.fori_loop(
        0, BSTEPS, step,
        (jnp.zeros((LANES,), jnp.int32),
         jnp.full((LANES,), ROWS_PER_W, jnp.int32)))

    bx_ref[pl.ds(0, LANES)] = b_v
    bx_ref[pl.ds(LANES, LANES)] = jnp.full((LANES,), ROWS_PER_W, jnp.int32)
    b_next = plsc.load_gather(bx_ref, [lane + 1])
    acc_ref[NUM_SEGMENTS, pl.ds(0, LANES)] = (b_next - b_v).astype(jnp.float32)

    for s in range(NUM_SEGMENTS):
        b_ref[s] = b_v[s]
    b_ref[NUM_SEGMENTS] = jnp.int32(ROWS_PER_W)

    for c in range(NCHUNK):
        cb = c & 1
        copies[cb].wait()

        def seg_body(s, _):
            lo = jnp.maximum(b_ref[s], c * CHUNK)
            hi = jnp.minimum(b_ref[s + 1], (c + 1) * CHUNK)

            def row_body(i, vs):
                r = i - c * CHUNK
                return tuple(
                    vs[k] + buf[cb, r, pl.ds(LANES * k, LANES)]
                    for k in range(KREG))

            vs = lax.fori_loop(
                lo, hi, row_body,
                tuple(jnp.zeros((LANES,), jnp.float32) for _ in range(KREG)))

            @pl.when(hi > lo)
            def _flush():
                for k in range(KREG):
                    acc_ref[s, pl.ds(LANES * k, LANES)] += vs[k]

            return 0

        lax.fori_loop(0, NUM_SEGMENTS, seg_body, 0)

        if c + 2 < NCHUNK:
            copies[cb] = pltpu.make_async_copy(
                flat_hbm.at[pl.ds(base + (c + 2) * CHUNK, CHUNK)], buf.at[cb],
                sems[cb])
            copies[cb].start()

    pltpu.sync_copy(acc_ref, psum_hbm.at[wid])


_sc_call = functools.partial(
    pl.kernel,
    out_type=jax.ShapeDtypeStruct((NWORKERS, NUM_SEGMENTS + 1, D), jnp.float32),
    mesh=plsc.VectorSubcoreMesh(core_axis_name="c", subcore_axis_name="s"),
    compiler_params=pltpu.CompilerParams(needs_layout_passes=False),
    scratch_types=[
        pltpu.VMEM((ROWS_PER_W,), jnp.int32),
        pltpu.SMEM((NUM_SEGMENTS + 1,), jnp.int32),
        pltpu.VMEM((2 * LANES,), jnp.int32),
        pltpu.VMEM((NUM_SEGMENTS + 1, D), jnp.float32),
        pltpu.VMEM((2, CHUNK, D), jnp.float32),
        pltpu.SemaphoreType.DMA,
        pltpu.SemaphoreType.DMA,
    ],
)


def _tc_body(ids_ref, flat_ref, psum_ref, pcnt_ref, acc_sum, acc_cnt):
    i = pl.program_id(0)

    @pl.when(i == 0)
    def _zero():
        acc_sum[...] = jnp.zeros_like(acc_sum)
        acc_cnt[...] = jnp.zeros_like(acc_cnt)

    ids = ids_ref[0, 0, :]
    seg_iota = jax.lax.broadcasted_iota(jnp.int32, (BLOCK_TOK, NUM_SEGMENTS), 1)
    onehot = (ids[:, None] == seg_iota).astype(jnp.float32)
    acc_sum[...] += jax.lax.dot_general(
        onehot, flat_ref[...],
        dimension_numbers=(((0,), (0,)), ((), ())),
        preferred_element_type=jnp.float32,
    )
    pcnt = jnp.sum(onehot, axis=0)
    acc_cnt[...] += jnp.broadcast_to(pcnt[:, None], (NUM_SEGMENTS, D))

    @pl.when(i == TC_GRID - 1)
    def _finish():
        psum_ref[...] = acc_sum[...]
        pcnt_ref[...] = acc_cnt[...]


def _combine_body(ps_ref, tps_ref, tpc_ref, out_ref):
    acc = jnp.sum(ps_ref[...], axis=0)
    sums = acc[:NUM_SEGMENTS, :] + tps_ref[...]
    cnts = acc[NUM_SEGMENTS, :NUM_SEGMENTS][:, None] + tpc_ref[...]
    out_ref[...] = sums / jnp.maximum(cnts, 1.0)


def kernel(flat, segment_ids):
    ids32 = segment_ids.astype(jnp.int32)
    psum = _sc_call(_sc_body)(flat, ids32)
    ids3 = ids32.reshape(TOTAL_TOK // BLOCK_TOK, 1, BLOCK_TOK)
    flat = pltpu.with_memory_space_constraint(flat, pltpu.MemorySpace.HBM)
    ids3 = pltpu.with_memory_space_constraint(ids3, pltpu.MemorySpace.HBM)
    tpsum, tpcnt = pl.pallas_call(
        _tc_body,
        grid=(TC_GRID,),
        in_specs=[
            pl.BlockSpec((1, 1, BLOCK_TOK), lambda i: (TC_OFF + i, 0, 0)),
            pl.BlockSpec((BLOCK_TOK, D), lambda i: (TC_OFF + i, 0)),
        ],
        out_specs=[
            pl.BlockSpec((NUM_SEGMENTS, D), lambda i: (0, 0)),
            pl.BlockSpec((NUM_SEGMENTS, D), lambda i: (0, 0)),
        ],
        out_shape=[
            jax.ShapeDtypeStruct((NUM_SEGMENTS, D), jnp.float32),
            jax.ShapeDtypeStruct((NUM_SEGMENTS, D), jnp.float32),
        ],
        scratch_shapes=[
            pltpu.VMEM((NUM_SEGMENTS, D), jnp.float32),
            pltpu.VMEM((NUM_SEGMENTS, D), jnp.float32),
        ],
    )(ids3, flat)
    return pl.pallas_call(
        _combine_body,
        out_shape=jax.ShapeDtypeStruct((NUM_SEGMENTS, D), jnp.float32),
    )(psum, tpsum, tpcnt)
